# baseline (speedup 1.0000x reference)
import functools

import jax
import jax.numpy as jnp
from jax.experimental import pallas as pl
from jax.experimental.pallas import tpu as pltpu

DIM = 768
TOPK = 2
N_EXPERT = 64
ROUTE_SCALE = 1.0
ALPHA = 0.1

BLOCK_ROWS = 1024
NSTREAM = 4


def _gate_kernel(*refs, nsteps, rows_per_step, n_tokens, nbatch):
    x_refs = refs[:NSTREAM]
    wt_ref = refs[NSTREAM]
    w_out_ref, i_out_ref, aux_ref, pi_acc, fi_acc = refs[NSTREAM + 1:]
    i = pl.program_id(0)

    @pl.when(i == 0)
    def _init():
        pi_acc[:] = jnp.zeros_like(pi_acc)
        fi_acc[:] = jnp.zeros_like(fi_acc)

    psum = jnp.zeros((N_EXPERT,), jnp.float32)
    cnt = jnp.zeros((N_EXPERT,), jnp.float32)
    for j, xr in enumerate(x_refs):
        s = jnp.dot(xr[:], wt_ref[:], preferred_element_type=jnp.float32)
        m = jnp.max(s, axis=-1, keepdims=True)
        e = jnp.exp(s - m)
        p = e / jnp.sum(e, axis=-1, keepdims=True)

        lane = jax.lax.broadcasted_iota(jnp.int32, p.shape, 1)
        a1 = jnp.argmax(p, axis=-1)
        v1 = jnp.max(p, axis=-1)
        m1 = lane == a1[:, None]
        pm = jnp.where(m1, -1.0, p)
        a2 = jnp.argmax(pm, axis=-1)
        v2 = jnp.max(pm, axis=-1)
        m2 = lane == a2[:, None]

        rows = pl.ds(j * BLOCK_ROWS, BLOCK_ROWS)
        w_out_ref[rows, :] = jnp.stack([v1, v2], axis=1) * ROUTE_SCALE
        i_out_ref[rows, :] = jnp.stack([a1, a2], axis=1)

        psum = psum + jnp.sum(p, axis=0)
        cnt = cnt + jnp.sum(jnp.where(m1 | m2, 1.0, 0.0), axis=0)

    batch = i // (n_tokens // rows_per_step)
    bh = (jax.lax.broadcasted_iota(jnp.int32, (nbatch, 1), 0) == batch
          ).astype(jnp.float32)
    pi_acc[:] += bh * psum[None, :]
    fi_acc[:] += bh * cnt[None, :]

    @pl.when(i == nsteps - 1)
    def _finish():
        fi = fi_acc[:] / (TOPK * n_tokens / N_EXPERT)
        pi = pi_acc[:] / n_tokens
        aux_ref[:, :] = jnp.sum(fi * pi, keepdims=True) * (ALPHA / nbatch)


def _mk_spec(j, d):
    return pl.BlockSpec((BLOCK_ROWS, d), lambda i, j=j: (NSTREAM * i + j, 0))


def kernel(x, W):
    b, n, d = x.shape
    xf = x.reshape(-1, d)
    rows = b * n
    rows_per_step = NSTREAM * BLOCK_ROWS
    nsteps = rows // rows_per_step
    wt = W.T

    body = functools.partial(
        _gate_kernel, nsteps=nsteps, rows_per_step=rows_per_step,
        n_tokens=n, nbatch=b)

    weight, idx, aux = pl.pallas_call(
        body,
        grid=(nsteps,),
        in_specs=[_mk_spec(j, d) for j in range(NSTREAM)] + [
            pl.BlockSpec((d, N_EXPERT), lambda i: (0, 0)),
        ],
        out_specs=[
            pl.BlockSpec((rows_per_step, TOPK), lambda i: (i, 0)),
            pl.BlockSpec((rows_per_step, TOPK), lambda i: (i, 0)),
            pl.BlockSpec((1, 1), lambda i: (0, 0)),
        ],
        out_shape=[
            jax.ShapeDtypeStruct((rows, TOPK), jnp.float32),
            jax.ShapeDtypeStruct((rows, TOPK), jnp.int32),
            jax.ShapeDtypeStruct((1, 1), jnp.float32),
        ],
        scratch_shapes=[
            pltpu.VMEM((b, N_EXPERT), jnp.float32),
            pltpu.VMEM((b, N_EXPERT), jnp.float32),
        ],
    )(*([xf] * NSTREAM), wt)
    return weight, idx, aux[0, 0]


# speedup vs baseline: 1.8692x; 1.8692x over previous
import functools

import jax
import jax.numpy as jnp
from jax.experimental import pallas as pl
from jax.experimental.pallas import tpu as pltpu

DIM = 768
TOPK = 2
N_EXPERT = 64
ROUTE_SCALE = 1.0
ALPHA = 0.1

BLOCK_ROWS = 1024
NSTREAM = 8


def _gate_kernel(*refs, nsteps, rows_per_step, n_tokens, nbatch):
    x_refs = refs[:NSTREAM]
    w_ref = refs[NSTREAM]
    w_out_ref, i_out_ref, aux_ref, pi_acc, fi_acc = refs[NSTREAM + 1:]
    i = pl.program_id(0)

    @pl.when(i == 0)
    def _init():
        pi_acc[:] = jnp.zeros_like(pi_acc)
        fi_acc[:] = jnp.zeros_like(fi_acc)

    p_loc = jnp.zeros((N_EXPERT, BLOCK_ROWS), jnp.float32)
    c_loc = jnp.zeros((N_EXPERT, BLOCK_ROWS), jnp.float32)
    for j, xr in enumerate(x_refs):
        s = jax.lax.dot_general(
            w_ref[:], xr[:], (((1,), (1,)), ((), ())),
            preferred_element_type=jnp.float32)
        m = jnp.max(s, axis=0, keepdims=True)
        e = jnp.exp(s - m)
        p = e / jnp.sum(e, axis=0, keepdims=True)

        sub = jax.lax.broadcasted_iota(jnp.int32, p.shape, 0)
        v1 = jnp.max(p, axis=0, keepdims=True)
        a1 = jnp.min(jnp.where(p == v1, sub, N_EXPERT), axis=0, keepdims=True)
        m1 = sub == a1
        pm = jnp.where(m1, -1.0, p)
        v2 = jnp.max(pm, axis=0, keepdims=True)
        a2 = jnp.min(jnp.where(pm == v2, sub, N_EXPERT), axis=0, keepdims=True)
        m2 = sub == a2

        cols = pl.ds(j * BLOCK_ROWS, BLOCK_ROWS)
        w_out_ref[:, cols] = jnp.concatenate([v1, v2], axis=0) * ROUTE_SCALE
        i_out_ref[:, cols] = jnp.concatenate([a1, a2], axis=0)

        p_loc = p_loc + p
        c_loc = c_loc + jnp.where(m1 | m2, 1.0, 0.0)

    pi_acc[i] += p_loc
    fi_acc[i] += c_loc

    @pl.when(i == nsteps - 1)
    def _finish():
        fi = jnp.sum(fi_acc[:], axis=2) / (TOPK * n_tokens / N_EXPERT)
        pi = jnp.sum(pi_acc[:], axis=2) / n_tokens
        aux_ref[:, :] = jnp.sum(fi * pi, keepdims=True) * (ALPHA / nbatch)


def _mk_spec(j, d):
    return pl.BlockSpec((BLOCK_ROWS, d), lambda i, j=j: (NSTREAM * i + j, 0))


def kernel(x, W):
    b, n, d = x.shape
    xf = x.reshape(-1, d)
    rows = b * n
    rows_per_step = NSTREAM * BLOCK_ROWS
    nsteps = rows // rows_per_step
    assert rows_per_step == n and nsteps == b

    body = functools.partial(
        _gate_kernel, nsteps=nsteps, rows_per_step=rows_per_step,
        n_tokens=n, nbatch=b)

    weight_t, idx_t, aux = pl.pallas_call(
        body,
        grid=(nsteps,),
        in_specs=[_mk_spec(j, d) for j in range(NSTREAM)] + [
            pl.BlockSpec((N_EXPERT, d), lambda i: (0, 0)),
        ],
        out_specs=[
            pl.BlockSpec((TOPK, rows_per_step), lambda i: (0, i)),
            pl.BlockSpec((TOPK, rows_per_step), lambda i: (0, i)),
            pl.BlockSpec((1, 1), lambda i: (0, 0)),
        ],
        out_shape=[
            jax.ShapeDtypeStruct((TOPK, rows), jnp.float32),
            jax.ShapeDtypeStruct((TOPK, rows), jnp.int32),
            jax.ShapeDtypeStruct((1, 1), jnp.float32),
        ],
        scratch_shapes=[
            pltpu.VMEM((b, N_EXPERT, BLOCK_ROWS), jnp.float32),
            pltpu.VMEM((b, N_EXPERT, BLOCK_ROWS), jnp.float32),
        ],
    )(*([xf] * NSTREAM), W)
    return weight_t.T, idx_t.T, aux[0, 0]
